# baseline (speedup 1.0000x reference)
import jax
import jax.numpy as jnp
from jax.experimental import pallas as pl

_PMIN = 0.1
_ROWS_PER_BLOCK = 32


def _mask_scale_kernel(x_ref, noise_ref, out_ref):
    p = jnp.float32(_PMIN)
    inv = jnp.float32(1.0) / (jnp.float32(1.0) - p)
    x = x_ref[...]
    noise = noise_ref[...]
    out_ref[...] = jnp.where(noise > p, x * inv, jnp.float32(0.0))


def kernel(x, noise):
    m, n = x.shape
    grid = (n // 8192,)
    spec = pl.BlockSpec((m, 8192), lambda i: (0, i))
    return pl.pallas_call(
        _mask_scale_kernel,
        grid=grid,
        in_specs=[spec, spec],
        out_specs=spec,
        out_shape=jax.ShapeDtypeStruct((m, n), jnp.float32),
    )(x, noise)


# speedup vs baseline: 1.0151x; 1.0151x over previous
import jax
import jax.numpy as jnp
from jax.experimental import pallas as pl

_PMIN = 0.1
_ROWS_PER_BLOCK = 32


def _mask_scale_kernel(x_ref, noise_ref, out_ref):
    p = jnp.float32(_PMIN)
    inv = jnp.float32(1.0) / (jnp.float32(1.0) - p)
    x = x_ref[...]
    noise = noise_ref[...]
    out_ref[...] = jnp.where(noise > p, x * inv, jnp.float32(0.0))


def kernel(x, noise):
    m, n = x.shape
    grid = (m // _ROWS_PER_BLOCK,)
    spec = pl.BlockSpec((_ROWS_PER_BLOCK, n), lambda i: (i, 0))
    return pl.pallas_call(
        _mask_scale_kernel,
        grid=grid,
        in_specs=[spec, spec],
        out_specs=spec,
        out_shape=jax.ShapeDtypeStruct((m, n), jnp.float32),
    )(x, noise)
